# baseline (speedup 1.0000x reference)
import functools

import jax
import jax.numpy as jnp
from jax import lax
from jax.experimental import pallas as pl
from jax.experimental.pallas import tpu as pltpu
from jax.experimental.pallas import tpu_sc as plsc

VOCAB = 1_000_000
D = 64
B = 4096 * 50
CHUNK = 512
NBINS = (VOCAB + CHUNK - 1) // CHUNK
LAST_BIN = NBINS - 1
LAST_START = LAST_BIN * CHUNK
NW = 32
NCL = 62
K = 192
BLK = 4096
NBLK = B // BLK
OUT_ROWS = B + 8


def _body(idx_hbm, tbl_hbm, tail_hbm, out_hbm, ibuf0, ibuf1, cbuf0, cbuf1,
          ebuf, aidx, apos, cidx, cpos, stag, counts, isem0, isem1, csem0,
          csem1):
    wid = lax.axis_index("s") * 2 + lax.axis_index("c")
    trash = B + (wid & 7)
    lanes = lax.iota(jnp.int32, 16)
    isem = (isem0, isem1)
    csem = (csem0, csem1)
    ibuf = (ibuf0, ibuf1)
    cbuf = (cbuf0, cbuf1)

    def czero(i, c):
        counts[i] = 0
        return c

    lax.fori_loop(0, NCL, czero, 0)

    def cstart(cl, par):
        bin_ = cl * 32 + wid

        @pl.when(bin_ <= LAST_BIN - 1)
        def _():
            pltpu.async_copy(tbl_hbm.at[:, pl.ds(bin_ * CHUNK, CHUNK)],
                             cbuf[par], csem[par])

    def cwait(cl, par):
        bin_ = cl * 32 + wid
        pltpu.make_async_copy(tbl_hbm.at[:, pl.ds(bin_ * CHUNK, CHUNK)],
                              cbuf[par], csem[par]).wait()

    cstart(0, 0)
    cstart(1, 1)

    def drain_groups(cbuf_ref, stage_start, n_real, n_groups, load_vecs):
        def group(g, c):
            base = g * 16
            ivec, pvec = load_vecs(base)
            valid = (base + lanes) < n_real
            ivec = jnp.where(valid, ivec, stage_start)
            pvec = jnp.where(valid, pvec, trash)
            lvec = ivec - stage_start
            for h in range(16):
                lv = jnp.full((16,), lvec[h], jnp.int32)
                for k in range(4):
                    stag[h, pl.ds(16 * k, 16)] = plsc.load_gather(
                        cbuf_ref, [lanes + (16 * k), lv])
            pltpu.sync_copy(stag, out_hbm.at[pvec])
            return c

        lax.fori_loop(0, n_groups, group, 0)

    def rescan(cbuf_ref, stage_start, lo, hi):
        def rblk(b, c):
            pltpu.sync_copy(idx_hbm.at[pl.ds(b * BLK, BLK)], ibuf0)

            def rvstep(v, cur):
                vec = ibuf0[pl.ds(v * 16, 16)]
                m = (vec >= lo) & (vec < hi)
                cnt = jnp.sum(m.astype(jnp.int32))
                plsc.store_compressed(aidx.at[pl.ds(cur, 16)], vec, mask=m)
                pvec = (b * BLK + v * 16) + lanes
                plsc.store_compressed(apos.at[pl.ds(cur, 16)], pvec, mask=m)
                return cur + cnt

            cur = lax.fori_loop(0, BLK // 16, rvstep, 0)
            drain_groups(cbuf_ref, stage_start, cur, (cur + 15) >> 4,
                         lambda base: (aidx[pl.ds(base, 16)],
                                       apos[pl.ds(base, 16)]))
            return c

        lax.fori_loop(0, NBLK, rblk, 0)

    def process_chunk(cl, cbuf_ref, stage_start, lo, hi):
        cnt = counts[cl]
        nstore = jnp.minimum(cnt, K)
        clv = jnp.full((16,), cl, jnp.int32)
        drain_groups(cbuf_ref, stage_start, nstore, (nstore + 15) >> 4,
                     lambda base: (plsc.load_gather(cidx, [clv, base + lanes]),
                                   plsc.load_gather(cpos, [clv, base + lanes])))

        @pl.when(cnt > K)
        def _():
            rescan(cbuf_ref, stage_start, lo, hi)

    pltpu.async_copy(idx_hbm.at[pl.ds(0, BLK)], ibuf0, isem[0])
    pltpu.async_copy(idx_hbm.at[pl.ds(BLK, BLK)], ibuf1, isem[1])

    def block_pair(p, c):
        for par in (0, 1):
            b = 2 * p + par
            pltpu.make_async_copy(idx_hbm.at[pl.ds(b * BLK, BLK)],
                                  ibuf[par], isem[par]).wait()

            def vstep(v, cur):
                vec = ibuf[par][pl.ds(v * 16, 16)]
                bb = lax.shift_right_logical(vec, 9)
                m = (bb & 31) == wid
                cnt = jnp.sum(m.astype(jnp.int32))
                plsc.store_compressed(aidx.at[pl.ds(cur, 16)], vec, mask=m)
                pvec = (b * BLK + v * 16) + lanes
                plsc.store_compressed(apos.at[pl.ds(cur, 16)], pvec, mask=m)
                return cur + cnt

            cur = lax.fori_loop(0, BLK // 16, vstep, 0)

            @pl.when(b + 2 < NBLK)
            def _():
                pltpu.async_copy(idx_hbm.at[pl.ds((b + 2) * BLK, BLK)],
                                 ibuf[par], isem[par])

            def bstep(g, cc):
                base = g * 16
                vec_i = aidx[pl.ds(base, 16)]
                vec_p = apos[pl.ds(base, 16)]
                cl_vec = lax.shift_right_logical(vec_i, 14)
                for h in range(16):
                    @pl.when(base + h < cur)
                    def _():
                        cl = cl_vec[h]
                        n = counts[cl]
                        onelane = lanes == h

                        @pl.when(n < K)
                        def _():
                            clv = jnp.full((16,), cl, jnp.int32)
                            nv = jnp.full((16,), n, jnp.int32)
                            plsc.store_scatter(cidx, [clv, nv], vec_i,
                                               mask=onelane)
                            plsc.store_scatter(cpos, [clv, nv], vec_p,
                                               mask=onelane)

                        counts[cl] = n + 1
                return cc

            lax.fori_loop(0, (cur + 15) >> 4, bstep, 0)
        return c

    lax.fori_loop(0, NBLK // 2, block_pair, 0)

    def pair(p, c):
        for par in (0, 1):
            cl = 2 * p + par
            bin_ = cl * 32 + wid

            @pl.when(bin_ <= LAST_BIN - 1)
            def _():
                cwait(cl, par)
                lo = bin_ * CHUNK
                process_chunk(cl, cbuf[par], lo, lo, lo + CHUNK)
                cstart(cl + 2, par)
        return c

    lax.fori_loop(0, NCL // 2, pair, 0)

    @pl.when(wid == (LAST_BIN & 31))
    def _():
        pltpu.sync_copy(tail_hbm, ebuf)
        process_chunk(LAST_BIN >> 5, ebuf, LAST_START, LAST_START, VOCAB)


@jax.jit
def _embed(idx_flat, tbl_t, tbl_tail):
    run = functools.partial(
        pl.kernel,
        mesh=plsc.VectorSubcoreMesh(core_axis_name="c", subcore_axis_name="s"),
        out_type=jax.ShapeDtypeStruct((OUT_ROWS, 2 * D), jnp.float32),
        scratch_types=[
            pltpu.VMEM((BLK,), jnp.int32),
            pltpu.VMEM((BLK,), jnp.int32),
            pltpu.VMEM((D, CHUNK), jnp.float32),
            pltpu.VMEM((D, CHUNK), jnp.float32),
            pltpu.VMEM((D, VOCAB - LAST_START), jnp.float32),
            pltpu.VMEM((BLK + 16,), jnp.int32),
            pltpu.VMEM((BLK + 16,), jnp.int32),
            pltpu.VMEM((NCL, K), jnp.int32),
            pltpu.VMEM((NCL, K), jnp.int32),
            pltpu.VMEM((16, 2 * D), jnp.float32),
            pltpu.SMEM((64,), jnp.int32),
            pltpu.SemaphoreType.DMA,
            pltpu.SemaphoreType.DMA,
            pltpu.SemaphoreType.DMA,
            pltpu.SemaphoreType.DMA,
        ],
        compiler_params=pltpu.CompilerParams(use_tc_tiling_on_sc=True,
                                             needs_layout_passes=False),
    )(_body)
    return run(idx_flat, tbl_t, tbl_tail)


def kernel(indice_sequence, embedding_matrix):
    idx_flat = indice_sequence.astype(jnp.int32).reshape(B)
    tbl_t = embedding_matrix.T
    out = _embed(idx_flat, tbl_t, tbl_t[:, LAST_START:])
    return out[:B, :D].reshape(indice_sequence.shape[0],
                               indice_sequence.shape[1], D)


# speedup vs baseline: 1.0731x; 1.0731x over previous
import functools

import jax
import jax.numpy as jnp
from jax import lax
from jax.experimental import pallas as pl
from jax.experimental.pallas import tpu as pltpu
from jax.experimental.pallas import tpu_sc as plsc

VOCAB = 1_000_000
D = 64
B = 4096 * 50
CHUNK = 512
NBINS = (VOCAB + CHUNK - 1) // CHUNK
LAST_BIN = NBINS - 1
LAST_START = LAST_BIN * CHUNK
TAIL_W = 128
TAIL_START = VOCAB - TAIL_W
NW = 32
NCL = 62
K = 176
BLK = 4096
QTR = BLK // 4 + 16
NBLK = B // BLK
OUT_ROWS = B + 8


def _body(idx_hbm, tbl_hbm, tail_hbm, out_hbm, ibuf0, ibuf1, cbuf0, cbuf1,
          aidx, apos, cidx, stag0, stag1, stag2, stag3, counts,
          isem0, isem1, csem0, csem1, ssem0, ssem1, ssem2, ssem3):
    wid = lax.axis_index("s") * 2 + lax.axis_index("c")
    trash = B + (wid & 7)
    lanes = lax.iota(jnp.int32, 16)
    isem = (isem0, isem1)
    csem = (csem0, csem1)
    ibuf = (ibuf0, ibuf1)
    cbuf = (cbuf0, cbuf1)
    stag = (stag0, stag1, stag2, stag3)
    ssem = (ssem0, ssem1, ssem2, ssem3)
    trash_vec = jnp.full((16,), B, jnp.int32) + (lanes & 7)

    def pcnt(m):
        return plsc.all_reduce_population_count(m)[0]

    def czero(i, c):
        counts[i] = 0
        return c

    lax.fori_loop(0, NCL, czero, 0)

    def cstart(cl, par):
        bin_ = cl * 32 + wid

        @pl.when(bin_ <= LAST_BIN - 1)
        def _():
            pltpu.async_copy(tbl_hbm.at[:, pl.ds(bin_ * CHUNK, CHUNK)],
                             cbuf[par], csem[par])

    def cwait(cl, par):
        bin_ = cl * 32 + wid
        pltpu.make_async_copy(tbl_hbm.at[:, pl.ds(bin_ * CHUNK, CHUNK)],
                              cbuf[par], csem[par]).wait()

    cstart(0, 0)
    cstart(1, 1)
    for q in range(4):
        pltpu.async_copy(stag[q], out_hbm.at[trash_vec], ssem[q])

    def drain_groups(cbuf_ref, loff, n_real, n_groups, load_vecs):
        def quad(Q, c):
            for q in range(4):
                g = Q * 4 + q

                @pl.when(g < n_groups)
                def _():
                    base = g * 16
                    packed = load_vecs(base)
                    valid = (base + lanes) < n_real
                    pvec = jnp.where(valid,
                                     lax.shift_right_logical(packed, 9), trash)
                    lvec = jnp.where(valid, (packed & (CHUNK - 1)) + loff, 0)
                    pltpu.make_async_copy(stag[q], out_hbm.at[pvec],
                                          ssem[q]).wait()
                    for d in range(D):
                        dv = jnp.full((16,), d, jnp.int32)
                        vals = plsc.load_gather(cbuf_ref, [dv, lvec])
                        plsc.store_scatter(stag[q], [lanes, dv], vals)
                    pltpu.async_copy(stag[q], out_hbm.at[pvec], ssem[q])
            return c

        lax.fori_loop(0, (n_groups + 3) >> 2, quad, 0)

    def rescan(cbuf_ref, loff, lo, hi):
        def rblk(b, c):
            pltpu.sync_copy(idx_hbm.at[pl.ds(b * BLK, BLK)], ibuf0)

            def rvstep(v, cur):
                vec = ibuf0[pl.ds(v * 16, 16)]
                m = (vec >= lo) & (vec < hi)
                pvec = (b * BLK + v * 16) + lanes
                packed = (vec & (CHUNK - 1)) | lax.shift_left(pvec, 9)
                plsc.store_compressed(aidx.at[pl.ds(cur, 16)], packed, mask=m)
                return cur + pcnt(m)

            cur = lax.fori_loop(0, BLK // 16, rvstep, 0)
            drain_groups(cbuf_ref, loff, cur, (cur + 15) >> 4,
                         lambda base: aidx[pl.ds(base, 16)])
            return c

        lax.fori_loop(0, NBLK, rblk, 0)

    def process_chunk(cl, cbuf_ref, loff, lo, hi):
        cnt = counts[cl]
        nstore = jnp.minimum(cnt, K)
        clv = jnp.full((16,), cl, jnp.int32)
        drain_groups(cbuf_ref, loff, nstore, (nstore + 15) >> 4,
                     lambda base: plsc.load_gather(cidx, [clv, base + lanes]))

        @pl.when(cnt > K)
        def _():
            rescan(cbuf_ref, loff, lo, hi)

    pltpu.async_copy(idx_hbm.at[pl.ds(0, BLK)], ibuf0, isem[0])
    pltpu.async_copy(idx_hbm.at[pl.ds(BLK, BLK)], ibuf1, isem[1])

    def block_pair(p, c):
        for par in (0, 1):
            b = 2 * p + par
            pltpu.make_async_copy(idx_hbm.at[pl.ds(b * BLK, BLK)],
                                  ibuf[par], isem[par]).wait()

            def vstep(v, curs):
                new = []
                for quarter in range(4):
                    cur = curs[quarter]
                    vv = 4 * v + quarter
                    vec = ibuf[par][pl.ds(vv * 16, 16)]
                    bb = lax.shift_right_logical(vec, 9)
                    m = (bb & 31) == wid
                    plsc.store_compressed(aidx.at[pl.ds(cur, 16)], vec,
                                          mask=m)
                    pvec = (b * BLK + vv * 16) + lanes
                    plsc.store_compressed(apos.at[pl.ds(cur, 16)], pvec,
                                          mask=m)
                    new.append(cur + pcnt(m))
                return tuple(new)

            curs = lax.fori_loop(0, BLK // 64, vstep,
                                 (0, QTR, 2 * QTR, 3 * QTR))

            @pl.when(b + 2 < NBLK)
            def _():
                pltpu.async_copy(idx_hbm.at[pl.ds((b + 2) * BLK, BLK)],
                                 ibuf[par], isem[par])

            def bstep_seg(seg_base, seg_cnt):
              def bstep(g, cc):
                base = seg_base + g * 16
                vec_i = aidx[pl.ds(base, 16)]
                vec_p = apos[pl.ds(base, 16)]
                packed = (vec_i & (CHUNK - 1)) | lax.shift_left(vec_p, 9)
                cl_vec = lax.shift_right_logical(vec_i, 14)
                for h in range(16):
                    @pl.when(base + h < seg_base + seg_cnt)
                    def _():
                        cl = cl_vec[h]
                        n = counts[cl]
                        clv = jnp.full((16,), cl, jnp.int32)
                        nv = jnp.full((16,), jnp.minimum(n, K - 1), jnp.int32)
                        plsc.store_scatter(cidx, [clv, nv], packed,
                                           mask=lanes == h)
                        counts[cl] = n + 1
                return cc
              lax.fori_loop(0, (seg_cnt + 15) >> 4, bstep, 0)

            for quarter in range(4):
                bstep_seg(quarter * QTR, curs[quarter] - quarter * QTR)
        return c

    with jax.named_scope("p1_filter_bin"):
        lax.fori_loop(0, NBLK // 2, block_pair, 0)

    def pair(p, c):
        for par in (0, 1):
            cl = 2 * p + par
            bin_ = cl * 32 + wid

            @pl.when(bin_ <= LAST_BIN - 1)
            def _():
                cwait(cl, par)
                lo = bin_ * CHUNK
                process_chunk(cl, cbuf[par], 0, lo, lo + CHUNK)
                cstart(cl + 2, par)
        return c

    with jax.named_scope("p2_chunks"):
        lax.fori_loop(0, NCL // 2, pair, 0)

    @pl.when(wid == (LAST_BIN & 31))
    def _():
        pltpu.sync_copy(tail_hbm, cbuf0.at[:, pl.ds(0, TAIL_W)])
        process_chunk(LAST_BIN >> 5, cbuf0, LAST_START - TAIL_START,
                      LAST_START, VOCAB)

    for q in range(4):
        pltpu.make_async_copy(stag[q], out_hbm.at[trash_vec], ssem[q]).wait()


@jax.jit
def _embed(idx_flat, tbl_t, tbl_tail):
    run = functools.partial(
        pl.kernel,
        mesh=plsc.VectorSubcoreMesh(core_axis_name="c", subcore_axis_name="s"),
        out_type=jax.ShapeDtypeStruct((OUT_ROWS, 2 * D), jnp.float32),
        scratch_types=[
            pltpu.VMEM((BLK,), jnp.int32),
            pltpu.VMEM((BLK,), jnp.int32),
            pltpu.VMEM((D, CHUNK), jnp.float32),
            pltpu.VMEM((D, CHUNK), jnp.float32),
            pltpu.VMEM((BLK + 64,), jnp.int32),
            pltpu.VMEM((BLK + 64,), jnp.int32),
            pltpu.VMEM((NCL, K), jnp.int32),
            pltpu.VMEM((16, 2 * D), jnp.float32),
            pltpu.VMEM((16, 2 * D), jnp.float32),
            pltpu.VMEM((16, 2 * D), jnp.float32),
            pltpu.VMEM((16, 2 * D), jnp.float32),
            pltpu.SMEM((64,), jnp.int32),
            pltpu.SemaphoreType.DMA,
            pltpu.SemaphoreType.DMA,
            pltpu.SemaphoreType.DMA,
            pltpu.SemaphoreType.DMA,
            pltpu.SemaphoreType.DMA,
            pltpu.SemaphoreType.DMA,
            pltpu.SemaphoreType.DMA,
            pltpu.SemaphoreType.DMA,
        ],
        compiler_params=pltpu.CompilerParams(use_tc_tiling_on_sc=True,
                                             needs_layout_passes=False),
    )(_body)
    return run(idx_flat, tbl_t, tbl_tail)


def kernel(indice_sequence, embedding_matrix):
    idx_flat = indice_sequence.astype(jnp.int32).reshape(B)
    tbl_t = embedding_matrix.T
    out = _embed(idx_flat, tbl_t, tbl_t[:, TAIL_START:])
    return out[:B, :D].reshape(indice_sequence.shape[0],
                               indice_sequence.shape[1], D)


# speedup vs baseline: 1.0811x; 1.0074x over previous
import functools

import jax
import jax.numpy as jnp
from jax import lax
from jax.experimental import pallas as pl
from jax.experimental.pallas import tpu as pltpu
from jax.experimental.pallas import tpu_sc as plsc

VOCAB = 1_000_000
D = 64
B = 4096 * 50
CHUNK = 512
NBINS = (VOCAB + CHUNK - 1) // CHUNK
LAST_BIN = NBINS - 1
LAST_START = LAST_BIN * CHUNK
TAIL_W = 128
TAIL_START = VOCAB - TAIL_W
NW = 32
NCL = 62
K = 176
BLK = 4096
QTR = BLK // 4 + 16
NBLK = B // BLK
OUT_ROWS = B + 8


def _body(idx_hbm, tbl_hbm, tail_hbm, out_hbm, ibuf0, ibuf1, cbuf0, cbuf1,
          aidx, apos, cidx, cpos, stag0, stag1, stag2, stag3, counts,
          isem0, isem1, csem0, csem1, ssem0, ssem1, ssem2, ssem3):
    wid = lax.axis_index("s") * 2 + lax.axis_index("c")
    trash = B + (wid & 7)
    lanes = lax.iota(jnp.int32, 16)
    isem = (isem0, isem1)
    csem = (csem0, csem1)
    ibuf = (ibuf0, ibuf1)
    cbuf = (cbuf0, cbuf1)
    stag = (stag0, stag1, stag2, stag3)
    ssem = (ssem0, ssem1, ssem2, ssem3)
    trash_vec = jnp.full((16,), B, jnp.int32) + (lanes & 7)

    def pcnt(m):
        return plsc.all_reduce_population_count(m)[0]

    def czero(i, c):
        counts[i] = 0
        return c

    lax.fori_loop(0, NCL, czero, 0)

    def cstart(cl, par):
        bin_ = cl * 32 + wid

        @pl.when(bin_ <= LAST_BIN - 1)
        def _():
            pltpu.async_copy(tbl_hbm.at[:, pl.ds(bin_ * CHUNK, CHUNK)],
                             cbuf[par], csem[par])

    def cwait(cl, par):
        bin_ = cl * 32 + wid
        pltpu.make_async_copy(tbl_hbm.at[:, pl.ds(bin_ * CHUNK, CHUNK)],
                              cbuf[par], csem[par]).wait()

    cstart(0, 0)
    cstart(1, 1)
    for q in range(4):
        pltpu.async_copy(stag[q], out_hbm.at[trash_vec], ssem[q])

    def drain_groups(cbuf_ref, stage_start, n_real, n_groups, load_vecs):
        def quad(Q, c):
            for q in range(4):
                g = Q * 4 + q

                @pl.when(g < n_groups)
                def _():
                    base = g * 16
                    ivec, pvec = load_vecs(base)
                    valid = (base + lanes) < n_real
                    ivec = jnp.where(valid, ivec, stage_start)
                    pvec = jnp.where(valid, pvec, trash)
                    lvec = ivec - stage_start
                    pltpu.make_async_copy(stag[q], out_hbm.at[pvec],
                                          ssem[q]).wait()
                    for d in range(D):
                        dv = jnp.full((16,), d, jnp.int32)
                        vals = plsc.load_gather(cbuf_ref, [dv, lvec])
                        plsc.store_scatter(stag[q], [lanes, dv], vals)
                    pltpu.async_copy(stag[q], out_hbm.at[pvec], ssem[q])
            return c

        lax.fori_loop(0, (n_groups + 3) >> 2, quad, 0)

    def rescan(cbuf_ref, stage_start, lo, hi):
        def rblk(b, c):
            pltpu.sync_copy(idx_hbm.at[pl.ds(b * BLK, BLK)], ibuf0)

            def rvstep(v, cur):
                vec = ibuf0[pl.ds(v * 16, 16)]
                m = (vec >= lo) & (vec < hi)
                plsc.store_compressed(aidx.at[pl.ds(cur, 16)], vec, mask=m)
                pvec = (b * BLK + v * 16) + lanes
                plsc.store_compressed(apos.at[pl.ds(cur, 16)], pvec, mask=m)
                return cur + pcnt(m)

            cur = lax.fori_loop(0, BLK // 16, rvstep, 0)
            drain_groups(cbuf_ref, stage_start, cur, (cur + 15) >> 4,
                         lambda base: (aidx[pl.ds(base, 16)],
                                       apos[pl.ds(base, 16)]))
            return c

        lax.fori_loop(0, NBLK, rblk, 0)

    def process_chunk(cl, cbuf_ref, stage_start, lo, hi):
        cnt = counts[cl]
        nstore = jnp.minimum(cnt, K)
        clv = jnp.full((16,), cl, jnp.int32)
        drain_groups(cbuf_ref, stage_start, nstore, (nstore + 15) >> 4,
                     lambda base: (plsc.load_gather(cidx, [clv, base + lanes]),
                                   plsc.load_gather(cpos, [clv, base + lanes])))

        @pl.when(cnt > K)
        def _():
            rescan(cbuf_ref, stage_start, lo, hi)

    pltpu.async_copy(idx_hbm.at[pl.ds(0, BLK)], ibuf0, isem[0])
    pltpu.async_copy(idx_hbm.at[pl.ds(BLK, BLK)], ibuf1, isem[1])

    def block_pair(p, c):
        for par in (0, 1):
            b = 2 * p + par
            pltpu.make_async_copy(idx_hbm.at[pl.ds(b * BLK, BLK)],
                                  ibuf[par], isem[par]).wait()

            def vstep(v, curs):
                new = []
                for quarter in range(4):
                    cur = curs[quarter]
                    vv = 4 * v + quarter
                    vec = ibuf[par][pl.ds(vv * 16, 16)]
                    bb = lax.shift_right_logical(vec, 9)
                    m = (bb & 31) == wid
                    plsc.store_compressed(aidx.at[pl.ds(cur, 16)], vec,
                                          mask=m)
                    pvec = (b * BLK + vv * 16) + lanes
                    plsc.store_compressed(apos.at[pl.ds(cur, 16)], pvec,
                                          mask=m)
                    new.append(cur + pcnt(m))
                return tuple(new)

            curs = lax.fori_loop(0, BLK // 64, vstep,
                                 (0, QTR, 2 * QTR, 3 * QTR))

            @pl.when(b + 2 < NBLK)
            def _():
                pltpu.async_copy(idx_hbm.at[pl.ds((b + 2) * BLK, BLK)],
                                 ibuf[par], isem[par])

            def bstep_seg(seg_base, seg_cnt):
              def bstep(g, cc):
                base = seg_base + g * 16
                vec_i = aidx[pl.ds(base, 16)]
                vec_p = apos[pl.ds(base, 16)]
                cl_vec = lax.shift_right_logical(vec_i, 14)
                for h in range(16):
                    @pl.when(base + h < seg_base + seg_cnt)
                    def _():
                        cl = cl_vec[h]
                        n = counts[cl]
                        onelane = lanes == h

                        @pl.when(n < K)
                        def _():
                            clv = jnp.full((16,), cl, jnp.int32)
                            nv = jnp.full((16,), n, jnp.int32)
                            plsc.store_scatter(cidx, [clv, nv], vec_i,
                                               mask=onelane)
                            plsc.store_scatter(cpos, [clv, nv], vec_p,
                                               mask=onelane)

                        counts[cl] = n + 1
                return cc
              lax.fori_loop(0, (seg_cnt + 15) >> 4, bstep, 0)

            for quarter in range(4):
                bstep_seg(quarter * QTR, curs[quarter] - quarter * QTR)
        return c

    with jax.named_scope("p1_filter_bin"):
        lax.fori_loop(0, NBLK // 2, block_pair, 0)

    def pair(p, c):
        for par in (0, 1):
            cl = 2 * p + par
            bin_ = cl * 32 + wid

            @pl.when(bin_ <= LAST_BIN - 1)
            def _():
                cwait(cl, par)
                lo = bin_ * CHUNK
                process_chunk(cl, cbuf[par], lo, lo, lo + CHUNK)
                cstart(cl + 2, par)
        return c

    with jax.named_scope("p2_chunks"):
        lax.fori_loop(0, NCL // 2, pair, 0)

    @pl.when(wid == (LAST_BIN & 31))
    def _():
        pltpu.sync_copy(tail_hbm, cbuf0.at[:, pl.ds(0, TAIL_W)])
        process_chunk(LAST_BIN >> 5, cbuf0, TAIL_START, LAST_START, VOCAB)

    for q in range(4):
        pltpu.make_async_copy(stag[q], out_hbm.at[trash_vec], ssem[q]).wait()


@jax.jit
def _embed(idx_flat, tbl_t, tbl_tail):
    run = functools.partial(
        pl.kernel,
        mesh=plsc.VectorSubcoreMesh(core_axis_name="c", subcore_axis_name="s"),
        out_type=jax.ShapeDtypeStruct((OUT_ROWS, 2 * D), jnp.float32),
        scratch_types=[
            pltpu.VMEM((BLK,), jnp.int32),
            pltpu.VMEM((BLK,), jnp.int32),
            pltpu.VMEM((D, CHUNK), jnp.float32),
            pltpu.VMEM((D, CHUNK), jnp.float32),
            pltpu.VMEM((BLK + 64,), jnp.int32),
            pltpu.VMEM((BLK + 64,), jnp.int32),
            pltpu.VMEM((NCL, K), jnp.int32),
            pltpu.VMEM((NCL, K), jnp.int32),
            pltpu.VMEM((16, 2 * D), jnp.float32),
            pltpu.VMEM((16, 2 * D), jnp.float32),
            pltpu.VMEM((16, 2 * D), jnp.float32),
            pltpu.VMEM((16, 2 * D), jnp.float32),
            pltpu.SMEM((64,), jnp.int32),
            pltpu.SemaphoreType.DMA,
            pltpu.SemaphoreType.DMA,
            pltpu.SemaphoreType.DMA,
            pltpu.SemaphoreType.DMA,
            pltpu.SemaphoreType.DMA,
            pltpu.SemaphoreType.DMA,
            pltpu.SemaphoreType.DMA,
            pltpu.SemaphoreType.DMA,
        ],
        compiler_params=pltpu.CompilerParams(use_tc_tiling_on_sc=True,
                                             needs_layout_passes=False),
    )(_body)
    return run(idx_flat, tbl_t, tbl_tail)


def kernel(indice_sequence, embedding_matrix):
    idx_flat = indice_sequence.astype(jnp.int32).reshape(B)
    tbl_t = embedding_matrix.T
    out = _embed(idx_flat, tbl_t, tbl_t[:, TAIL_START:])
    return out[:B, :D].reshape(indice_sequence.shape[0],
                               indice_sequence.shape[1], D)
